# baseline (speedup 1.0000x reference)
import functools

import jax
import jax.numpy as jnp
from jax.experimental import pallas as pl
from jax.experimental.pallas import tpu as pltpu
from jax.experimental.pallas import tpu_sc as plsc

TM = 64
BH = 1024


def _routing_body(x_ref, gw_ref, pos0_ref, pos1_ref, w0_ref, w1_ref, counts_ref):
    x = x_ref[...]
    gw = gw_ref[...]
    logits = jax.lax.dot_general(
        x, gw, (((1,), (1,)), ((), ())), preferred_element_type=jnp.float32
    )
    t, e = logits.shape
    neg = jnp.float32(-1e30)
    eidx = jax.lax.broadcasted_iota(jnp.int32, (t, e), 1)
    m0 = jnp.max(logits, axis=1, keepdims=True)
    a0 = jnp.min(jnp.where(logits == m0, eidx, e), axis=1, keepdims=True)
    oh0 = eidx == a0
    masked = jnp.where(oh0, neg, logits)
    m1 = jnp.max(masked, axis=1, keepdims=True)
    a1 = jnp.min(jnp.where(masked == m1, eidx, e), axis=1, keepdims=True)
    oh1 = eidx == a1
    z = jnp.exp(m1 - m0)
    denom = 1.0 + z
    w0_ref[...] = (1.0 / denom).astype(jnp.float32)
    w1_ref[...] = (z / denom).astype(jnp.float32)
    both = oh0.astype(jnp.float32) + oh1.astype(jnp.float32)
    blk = 128
    ridx = jax.lax.broadcasted_iota(jnp.int32, (blk, t), 1)
    excl_blocks = []
    for c in range(t // blk):
        rows = c * blk + jax.lax.broadcasted_iota(jnp.int32, (blk, t), 0)
        tri = (ridx < rows).astype(jnp.float32)
        excl_blocks.append(
            jax.lax.dot_general(
                tri, both, (((1,), (0,)), ((), ())),
                preferred_element_type=jnp.float32,
            )
        )
    excl = jnp.concatenate(excl_blocks, axis=0)
    counts = jnp.sum(both, axis=0, keepdims=True)
    ce = jax.lax.broadcasted_iota(jnp.int32, (e, e), 0)
    re_ = jax.lax.broadcasted_iota(jnp.int32, (e, e), 1)
    tril_e = (ce < re_).astype(jnp.float32)
    starts = jax.lax.dot_general(
        counts, tril_e, (((1,), (0,)), ((), ())),
        preferred_element_type=jnp.float32,
    )
    base = starts + excl
    pos0 = jnp.sum(jnp.where(oh0, base, 0.0), axis=1)
    pos1 = jnp.sum(jnp.where(oh1, base, 0.0), axis=1)
    pos0_ref[...] = pos0[:, None].astype(jnp.int32)
    pos1_ref[...] = pos1[:, None].astype(jnp.int32)
    counts_ref[...] = counts.astype(jnp.int32)


def _routing(x, gate_w):
    t, _ = x.shape
    e = gate_w.shape[0]
    return pl.pallas_call(
        _routing_body,
        out_shape=(
            jax.ShapeDtypeStruct((t, 1), jnp.int32),
            jax.ShapeDtypeStruct((t, 1), jnp.int32),
            jax.ShapeDtypeStruct((t, 1), jnp.float32),
            jax.ShapeDtypeStruct((t, 1), jnp.float32),
            jax.ShapeDtypeStruct((1, e), jnp.int32),
        ),
    )(x, gate_w)


def _gmm_body(ti_ref, ei_ref, rs_ref, re_ref, xs_ref, w1_ref, w2_ref, w3_ref, out_ref):
    u = pl.program_id(0)
    rs = rs_ref[u]
    re = re_ref[u]

    @pl.when(re > rs)
    def _():
        row0 = ti_ref[u] * TM
        gid = row0 + jax.lax.broadcasted_iota(jnp.int32, (TM, 1), 0)
        mask = (gid >= rs) & (gid < re)
        x_t = xs_ref[...]
        w1c = w1_ref[0]
        w3c = w3_ref[0]
        w2c = w2_ref[0]
        xw1 = jax.lax.dot_general(
            x_t, w1c, (((1,), (1,)), ((), ())), preferred_element_type=jnp.float32
        )
        xw3 = jax.lax.dot_general(
            x_t, w3c, (((1,), (1,)), ((), ())), preferred_element_type=jnp.float32
        )
        h = (xw1 * jax.nn.sigmoid(xw1)) * xw3
        o = jax.lax.dot_general(
            h, w2c, (((1,), (0,)), ((), ())), preferred_element_type=jnp.float32
        )
        out_ref[...] = jnp.where(mask, o, out_ref[...])


def _gmm(xs, w1, w2, w3, ti, ei, rs, re, num_units):
    s, d = xs.shape
    e, h, _ = w1.shape
    wspec = pl.BlockSpec((1, h, d), lambda u, ti, ei, rs, re: (ei[u], 0, 0))
    rowspec = pl.BlockSpec((TM, d), lambda u, ti, ei, rs, re: (ti[u], 0))
    return pl.pallas_call(
        _gmm_body,
        grid_spec=pltpu.PrefetchScalarGridSpec(
            num_scalar_prefetch=4,
            grid=(num_units,),
            in_specs=[rowspec, wspec, wspec, wspec],
            out_specs=rowspec,
        ),
        out_shape=jax.ShapeDtypeStruct((s, d), jnp.float32),
    )(ti, ei, rs, re, xs, w1, w2, w3)


def _combine_body(g0_ref, g1_ref, w0_ref, w1_ref, y_ref):
    y_ref[...] = g0_ref[...] * w0_ref[...] + g1_ref[...] * w1_ref[...]


def _combine(g0, g1, w0, w1):
    t, d = g0.shape
    tmc = 256
    return pl.pallas_call(
        _combine_body,
        grid=(t // tmc,),
        in_specs=[
            pl.BlockSpec((tmc, d), lambda i: (i, 0)),
            pl.BlockSpec((tmc, d), lambda i: (i, 0)),
            pl.BlockSpec((tmc, 1), lambda i: (i, 0)),
            pl.BlockSpec((tmc, 1), lambda i: (i, 0)),
        ],
        out_specs=pl.BlockSpec((tmc, d), lambda i: (i, 0)),
        out_shape=jax.ShapeDtypeStruct((t, d), jnp.float32),
    )(g0, g1, w0, w1)


_NC = 2
_NS = 16
_NW = _NC * _NS


def _sc_dispatch(x, p0, p1, s):
    t, d = x.shape
    bpw = t // _NW
    mesh = plsc.VectorSubcoreMesh(core_axis_name="c", subcore_axis_name="s")

    @functools.partial(
        pl.kernel,
        mesh=mesh,
        out_type=jax.ShapeDtypeStruct((s, d), jnp.float32),
        scratch_types=[
            pltpu.VMEM((bpw,), jnp.int32),
            pltpu.VMEM((bpw,), jnp.int32),
            pltpu.VMEM((bpw, d), jnp.float32),
            pltpu.SemaphoreType.DMA,
        ],
    )
    def k(x_hbm, p0_hbm, p1_hbm, o_hbm, i0_v, i1_v, rows_v, sem):
        wid = jax.lax.axis_index("s") * _NC + jax.lax.axis_index("c")
        base = wid * bpw
        pltpu.sync_copy(p0_hbm.at[pl.ds(base, bpw)], i0_v)
        pltpu.sync_copy(p1_hbm.at[pl.ds(base, bpw)], i1_v)
        pltpu.sync_copy(x_hbm.at[pl.ds(base, bpw)], rows_v)
        pltpu.async_copy(rows_v, o_hbm.at[i0_v], sem).wait()
        pltpu.async_copy(rows_v, o_hbm.at[i1_v], sem).wait()

    return k(x, p0, p1)


def _sc_gather2(os_, p0, p1):
    s, d = os_.shape
    t = p0.shape[0]
    bpw = t // _NW
    mesh = plsc.VectorSubcoreMesh(core_axis_name="c", subcore_axis_name="s")
    ot = jax.ShapeDtypeStruct((t, d), jnp.float32)

    @functools.partial(
        pl.kernel,
        mesh=mesh,
        out_type=(ot, ot),
        scratch_types=[
            pltpu.VMEM((bpw,), jnp.int32),
            pltpu.VMEM((bpw, d), jnp.float32),
            pltpu.SemaphoreType.DMA,
        ],
    )
    def k(os_hbm, p0_hbm, p1_hbm, g0_hbm, g1_hbm, idx_v, rows_v, sem):
        wid = jax.lax.axis_index("s") * _NC + jax.lax.axis_index("c")
        base = wid * bpw
        pltpu.sync_copy(p0_hbm.at[pl.ds(base, bpw)], idx_v)
        pltpu.async_copy(os_hbm.at[idx_v], rows_v, sem).wait()
        pltpu.sync_copy(rows_v, g0_hbm.at[pl.ds(base, bpw)])
        pltpu.sync_copy(p1_hbm.at[pl.ds(base, bpw)], idx_v)
        pltpu.async_copy(os_hbm.at[idx_v], rows_v, sem).wait()
        pltpu.sync_copy(rows_v, g1_hbm.at[pl.ds(base, bpw)])

    return k(os_, p0, p1)


def _schedule(counts, num_tiles, w_static):
    e = counts.shape[0]
    starts = jnp.cumsum(counts) - counts
    ends = starts + counts
    first = starts // TM
    last = jnp.where(counts > 0, (ends - 1) // TM, first)
    nt = jnp.where(counts > 0, last - first + 1, 0)
    us = jnp.cumsum(nt) - nt
    total = jnp.sum(nt)
    u = jnp.arange(w_static, dtype=jnp.int32)
    e_of = jnp.clip(jnp.searchsorted(us, u, side="right") - 1, 0, e - 1).astype(
        jnp.int32
    )
    ti = (first[e_of] + (u - us[e_of])).astype(jnp.int32)
    rs = starts[e_of].astype(jnp.int32)
    re = ends[e_of].astype(jnp.int32)
    valid = u < total
    last_u = total - 1
    ti = jnp.where(valid, ti, ti[last_u])
    e_of = jnp.where(valid, e_of, e_of[last_u])
    rs = jnp.where(valid, rs, 0)
    re = jnp.where(valid, re, 0)
    fv = jnp.concatenate(
        [jnp.ones((1,), jnp.int32), (jnp.diff(ti) != 0).astype(jnp.int32)]
    )
    fv = jnp.where(valid, fv, 0)
    return ti, e_of, rs, re, fv


def kernel(x, gate_w, w1, w2, w3):
    t, d = x.shape
    e = gate_w.shape[0]
    k = 2
    s = t * k
    num_tiles = s // TM
    w_static = num_tiles + e - 1

    pos0, pos1, wt0, wt1, counts2d = _routing(x, gate_w)
    p0r = pos0.reshape(t)
    p1r = pos1.reshape(t)
    ti, ei, rs, re, fv = _schedule(counts2d[0], num_tiles, w_static)
    del fv

    xs = _sc_dispatch(x, p0r, p1r, s)
    os = _gmm(xs, w1, w2, w3, ti, ei, rs, re, w_static)
    g0, g1 = _sc_gather2(os, p0r, p1r)
    return _combine(g0, g1, wt0, wt1)


# speedup vs baseline: 1.0974x; 1.0974x over previous
import functools

import jax
import jax.numpy as jnp
from jax.experimental import pallas as pl
from jax.experimental.pallas import tpu as pltpu
from jax.experimental.pallas import tpu_sc as plsc

TM = 128
BH = 1024


def _routing_body(
    x_ref, gw_ref, pos0_ref, pos1_ref, w0_ref, w1_ref,
    ti_ref, ei_ref, rs_ref, re_ref,
):
    x = x_ref[...]
    gw = gw_ref[...]
    logits = jax.lax.dot_general(
        x, gw, (((1,), (1,)), ((), ())), preferred_element_type=jnp.float32
    )
    t, e = logits.shape
    neg = jnp.float32(-1e30)
    eidx = jax.lax.broadcasted_iota(jnp.int32, (t, e), 1)
    m0 = jnp.max(logits, axis=1, keepdims=True)
    a0 = jnp.min(jnp.where(logits == m0, eidx, e), axis=1, keepdims=True)
    oh0 = eidx == a0
    masked = jnp.where(oh0, neg, logits)
    m1 = jnp.max(masked, axis=1, keepdims=True)
    a1 = jnp.min(jnp.where(masked == m1, eidx, e), axis=1, keepdims=True)
    oh1 = eidx == a1
    z = jnp.exp(m1 - m0)
    denom = 1.0 + z
    w0_ref[...] = (1.0 / denom).astype(jnp.float32)
    w1_ref[...] = (z / denom).astype(jnp.float32)
    both = oh0.astype(jnp.float32) + oh1.astype(jnp.float32)
    blk = 128
    ridx = jax.lax.broadcasted_iota(jnp.int32, (blk, t), 1)
    excl_blocks = []
    for c in range(t // blk):
        rows = c * blk + jax.lax.broadcasted_iota(jnp.int32, (blk, t), 0)
        tri = (ridx < rows).astype(jnp.float32)
        excl_blocks.append(
            jax.lax.dot_general(
                tri, both, (((1,), (0,)), ((), ())),
                preferred_element_type=jnp.float32,
            )
        )
    excl = jnp.concatenate(excl_blocks, axis=0)
    counts = jnp.sum(both, axis=0, keepdims=True)
    ce = jax.lax.broadcasted_iota(jnp.int32, (e, e), 0)
    re_ = jax.lax.broadcasted_iota(jnp.int32, (e, e), 1)
    tril_e = (ce < re_).astype(jnp.float32)
    starts = jax.lax.dot_general(
        counts, tril_e, (((1,), (0,)), ((), ())),
        preferred_element_type=jnp.float32,
    )
    base = starts + excl
    pos0 = jnp.sum(jnp.where(oh0, base, 0.0), axis=1)
    pos1 = jnp.sum(jnp.where(oh1, base, 0.0), axis=1)
    pos0_ref[...] = pos0[:, None].astype(jnp.int32)
    pos1_ref[...] = pos1[:, None].astype(jnp.int32)

    wpad = ti_ref.shape[0]
    ends = starts + counts
    inv_tm = jnp.float32(1.0 / TM)
    first = jnp.floor(starts * inv_tm)
    last = jnp.where(counts > 0, jnp.floor((ends - 1.0) * inv_tm), first)
    nt = jnp.where(counts > 0, last - first + 1.0, 0.0)
    us = jax.lax.dot_general(
        nt, tril_e, (((1,), (0,)), ((), ())),
        preferred_element_type=jnp.float32,
    )
    total = jnp.sum(nt)
    uvec = jax.lax.broadcasted_iota(jnp.int32, (wpad, 1), 0).astype(jnp.float32)
    cmp = us <= uvec
    e_of = jnp.sum(cmp.astype(jnp.float32), axis=1, keepdims=True) - 1.0
    eidw = jax.lax.broadcasted_iota(jnp.int32, (wpad, e), 1).astype(jnp.float32)
    ohw = (eidw == e_of).astype(jnp.float32)
    first_u = jnp.sum(ohw * first, axis=1, keepdims=True)
    us_u = jnp.sum(ohw * us, axis=1, keepdims=True)
    rs_u = jnp.sum(ohw * starts, axis=1, keepdims=True)
    re_u = jnp.sum(ohw * ends, axis=1, keepdims=True)
    ti_u = first_u + (uvec - us_u)
    valid = uvec < total
    lastmask = (uvec == total - 1.0).astype(jnp.float32)
    ti_last = jnp.sum(lastmask * ti_u)
    ei_last = jnp.sum(lastmask * e_of)
    ti_u = jnp.where(valid, ti_u, ti_last)
    e_of = jnp.where(valid, e_of, ei_last)
    rs_u = jnp.where(valid, rs_u, 0.0)
    re_u = jnp.where(valid, re_u, 0.0)
    ti_ref[...] = ti_u.astype(jnp.int32)
    ei_ref[...] = e_of.astype(jnp.int32)
    rs_ref[...] = rs_u.astype(jnp.int32)
    re_ref[...] = re_u.astype(jnp.int32)


def _routing(x, gate_w, wpad):
    t, _ = x.shape
    e = gate_w.shape[0]
    return pl.pallas_call(
        _routing_body,
        out_shape=(
            jax.ShapeDtypeStruct((t, 1), jnp.int32),
            jax.ShapeDtypeStruct((t, 1), jnp.int32),
            jax.ShapeDtypeStruct((t, 1), jnp.float32),
            jax.ShapeDtypeStruct((t, 1), jnp.float32),
            jax.ShapeDtypeStruct((wpad, 1), jnp.int32),
            jax.ShapeDtypeStruct((wpad, 1), jnp.int32),
            jax.ShapeDtypeStruct((wpad, 1), jnp.int32),
            jax.ShapeDtypeStruct((wpad, 1), jnp.int32),
        ),
    )(x, gate_w)


def _gmm_body(ti_ref, ei_ref, rs_ref, re_ref, xs_ref, w1_ref, w2_ref, w3_ref, out_ref):
    u = pl.program_id(1)
    rs = rs_ref[u]
    re = re_ref[u]

    @pl.when(re > rs)
    def _():
        row0 = ti_ref[u] * TM
        gid = row0 + jax.lax.broadcasted_iota(jnp.int32, (TM, 1), 0)
        mask = (gid >= rs) & (gid < re)
        x_t = xs_ref[...].astype(jnp.bfloat16)
        w1c = w1_ref[0].astype(jnp.bfloat16)
        w3c = w3_ref[0].astype(jnp.bfloat16)
        w2c = w2_ref[0].astype(jnp.bfloat16)
        xw1 = jax.lax.dot_general(
            x_t, w1c, (((1,), (1,)), ((), ())), preferred_element_type=jnp.float32
        )
        xw3 = jax.lax.dot_general(
            x_t, w3c, (((1,), (1,)), ((), ())), preferred_element_type=jnp.float32
        )
        h = (xw1 * jax.nn.sigmoid(xw1)) * xw3
        o = jax.lax.dot_general(
            h.astype(jnp.bfloat16), w2c,
            (((1,), (0,)), ((), ())), preferred_element_type=jnp.float32
        )
        out_ref[0] = jnp.where(mask, o, out_ref[0])


def _gmm(xs, w1, w2, w3, ti, ei, rs, re, num_units):
    s, d = xs.shape
    e, h, _ = w1.shape
    nhj = h // BH
    wspec = pl.BlockSpec((1, BH, d), lambda hj, u, ti, ei, rs, re: (ei[u], hj, 0))
    xspec = pl.BlockSpec((TM, d), lambda hj, u, ti, ei, rs, re: (ti[u], 0))
    ospec = pl.BlockSpec((1, TM, d), lambda hj, u, ti, ei, rs, re: (hj, ti[u], 0))
    return pl.pallas_call(
        _gmm_body,
        grid_spec=pltpu.PrefetchScalarGridSpec(
            num_scalar_prefetch=4,
            grid=(nhj, num_units),
            in_specs=[xspec, wspec, wspec, wspec],
            out_specs=ospec,
        ),
        out_shape=jax.ShapeDtypeStruct((nhj, s, d), jnp.float32),
    )(ti, ei, rs, re, xs, w1, w2, w3)


def _sum2_body(a_ref, o_ref):
    o_ref[...] = a_ref[0] + a_ref[1]


def _sum2(osf):
    nhj, s, d = osf.shape
    tmc = 512
    return pl.pallas_call(
        _sum2_body,
        grid=(s // tmc,),
        in_specs=[pl.BlockSpec((nhj, tmc, d), lambda i: (0, i, 0))],
        out_specs=pl.BlockSpec((tmc, d), lambda i: (i, 0)),
        out_shape=jax.ShapeDtypeStruct((s, d), jnp.float32),
    )(osf)


def _combine_body(g0_ref, g1_ref, w0_ref, w1_ref, y_ref):
    y_ref[...] = g0_ref[...] * w0_ref[...] + g1_ref[...] * w1_ref[...]


def _combine(g0, g1, w0, w1):
    t, d = g0.shape
    tmc = 256
    return pl.pallas_call(
        _combine_body,
        grid=(t // tmc,),
        in_specs=[
            pl.BlockSpec((tmc, d), lambda i: (i, 0)),
            pl.BlockSpec((tmc, d), lambda i: (i, 0)),
            pl.BlockSpec((tmc, 1), lambda i: (i, 0)),
            pl.BlockSpec((tmc, 1), lambda i: (i, 0)),
        ],
        out_specs=pl.BlockSpec((tmc, d), lambda i: (i, 0)),
        out_shape=jax.ShapeDtypeStruct((t, d), jnp.float32),
    )(g0, g1, w0, w1)


_NC = 2
_NS = 16
_NW = _NC * _NS


def _sc_dispatch(x, p0, p1, s):
    t, d = x.shape
    bpw = t // _NW
    mesh = plsc.VectorSubcoreMesh(core_axis_name="c", subcore_axis_name="s")

    @functools.partial(
        pl.kernel,
        mesh=mesh,
        out_type=jax.ShapeDtypeStruct((s, d), jnp.float32),
        scratch_types=[
            pltpu.VMEM((bpw,), jnp.int32),
            pltpu.VMEM((bpw,), jnp.int32),
            pltpu.VMEM((bpw, d), jnp.float32),
            pltpu.SemaphoreType.DMA,
        ],
    )
    def k(x_hbm, p0_hbm, p1_hbm, o_hbm, i0_v, i1_v, rows_v, sem):
        wid = jax.lax.axis_index("s") * _NC + jax.lax.axis_index("c")
        base = wid * bpw
        pltpu.sync_copy(p0_hbm.at[pl.ds(base, bpw)], i0_v)
        pltpu.sync_copy(p1_hbm.at[pl.ds(base, bpw)], i1_v)
        pltpu.sync_copy(x_hbm.at[pl.ds(base, bpw)], rows_v)
        pltpu.async_copy(rows_v, o_hbm.at[i0_v], sem).wait()
        pltpu.async_copy(rows_v, o_hbm.at[i1_v], sem).wait()

    return k(x, p0, p1)


def _sc_gather2(os_, p0, p1):
    s, d = os_.shape
    t = p0.shape[0]
    bpw = t // _NW
    mesh = plsc.VectorSubcoreMesh(core_axis_name="c", subcore_axis_name="s")
    ot = jax.ShapeDtypeStruct((t, d), jnp.float32)

    @functools.partial(
        pl.kernel,
        mesh=mesh,
        out_type=(ot, ot),
        scratch_types=[
            pltpu.VMEM((bpw,), jnp.int32),
            pltpu.VMEM((bpw, d), jnp.float32),
            pltpu.SemaphoreType.DMA,
        ],
    )
    def k(os_hbm, p0_hbm, p1_hbm, g0_hbm, g1_hbm, idx_v, rows_v, sem):
        wid = jax.lax.axis_index("s") * _NC + jax.lax.axis_index("c")
        base = wid * bpw
        pltpu.sync_copy(p0_hbm.at[pl.ds(base, bpw)], idx_v)
        pltpu.async_copy(os_hbm.at[idx_v], rows_v, sem).wait()
        pltpu.sync_copy(rows_v, g0_hbm.at[pl.ds(base, bpw)])
        pltpu.sync_copy(p1_hbm.at[pl.ds(base, bpw)], idx_v)
        pltpu.async_copy(os_hbm.at[idx_v], rows_v, sem).wait()
        pltpu.sync_copy(rows_v, g1_hbm.at[pl.ds(base, bpw)])

    return k(os_, p0, p1)


def kernel(x, gate_w, w1, w2, w3):
    t, d = x.shape
    e = gate_w.shape[0]
    k = 2
    s = t * k
    num_tiles = s // TM
    w_static = num_tiles + e - 1

    wpad = -(-w_static // 8) * 8

    pos0, pos1, wt0, wt1, ti2, ei2, rs2, re2 = _routing(x, gate_w, wpad)
    p0r = pos0.reshape(t)
    p1r = pos1.reshape(t)
    ti = ti2.reshape(wpad)[:w_static]
    ei = ei2.reshape(wpad)[:w_static]
    rs = rs2.reshape(wpad)[:w_static]
    re = re2.reshape(wpad)[:w_static]

    xs = _sc_dispatch(x, p0r, p1r, s)
    osf = _gmm(xs, w1, w2, w3, ti, ei, rs, re, w_static)
    os = _sum2(osf)
    g0, g1 = _sc_gather2(os, p0r, p1r)
    return _combine(g0, g1, wt0, wt1)


# speedup vs baseline: 1.2132x; 1.1055x over previous
import functools

import jax
import jax.numpy as jnp
from jax.experimental import pallas as pl
from jax.experimental.pallas import tpu as pltpu
from jax.experimental.pallas import tpu_sc as plsc

TM = 256
BH = 1024


def _routing_body(
    x_ref, gw_ref, pos0_ref, pos1_ref, w0_ref, w1_ref,
    ti_ref, ei_ref, rs_ref, re_ref,
):
    x = x_ref[...]
    gw = gw_ref[...]
    logits = jax.lax.dot_general(
        x, gw, (((1,), (1,)), ((), ())), preferred_element_type=jnp.float32
    )
    t, e = logits.shape
    neg = jnp.float32(-1e30)
    eidx = jax.lax.broadcasted_iota(jnp.int32, (t, e), 1)
    m0 = jnp.max(logits, axis=1, keepdims=True)
    a0 = jnp.min(jnp.where(logits == m0, eidx, e), axis=1, keepdims=True)
    oh0 = eidx == a0
    masked = jnp.where(oh0, neg, logits)
    m1 = jnp.max(masked, axis=1, keepdims=True)
    a1 = jnp.min(jnp.where(masked == m1, eidx, e), axis=1, keepdims=True)
    oh1 = eidx == a1
    z = jnp.exp(m1 - m0)
    denom = 1.0 + z
    w0_ref[...] = (1.0 / denom).astype(jnp.float32)
    w1_ref[...] = (z / denom).astype(jnp.float32)
    both = oh0.astype(jnp.float32) + oh1.astype(jnp.float32)
    blk = 128
    ridx = jax.lax.broadcasted_iota(jnp.int32, (blk, t), 1)
    excl_blocks = []
    for c in range(t // blk):
        rows = c * blk + jax.lax.broadcasted_iota(jnp.int32, (blk, t), 0)
        tri = (ridx < rows).astype(jnp.float32)
        excl_blocks.append(
            jax.lax.dot_general(
                tri, both, (((1,), (0,)), ((), ())),
                preferred_element_type=jnp.float32,
            )
        )
    excl = jnp.concatenate(excl_blocks, axis=0)
    counts = jnp.sum(both, axis=0, keepdims=True)
    ce = jax.lax.broadcasted_iota(jnp.int32, (e, e), 0)
    re_ = jax.lax.broadcasted_iota(jnp.int32, (e, e), 1)
    tril_e = (ce < re_).astype(jnp.float32)
    starts = jax.lax.dot_general(
        counts, tril_e, (((1,), (0,)), ((), ())),
        preferred_element_type=jnp.float32,
    )
    base = starts + excl
    pos0 = jnp.sum(jnp.where(oh0, base, 0.0), axis=1)
    pos1 = jnp.sum(jnp.where(oh1, base, 0.0), axis=1)
    pos0_ref[...] = pos0[:, None].astype(jnp.int32)
    pos1_ref[...] = pos1[:, None].astype(jnp.int32)

    wpad = ti_ref.shape[0]
    ends = starts + counts
    inv_tm = jnp.float32(1.0 / TM)
    first = jnp.floor(starts * inv_tm)
    last = jnp.where(counts > 0, jnp.floor((ends - 1.0) * inv_tm), first)
    nt = jnp.where(counts > 0, last - first + 1.0, 0.0)
    us = jax.lax.dot_general(
        nt, tril_e, (((1,), (0,)), ((), ())),
        preferred_element_type=jnp.float32,
    )
    total = jnp.sum(nt)
    uvec = jax.lax.broadcasted_iota(jnp.int32, (wpad, 1), 0).astype(jnp.float32)
    cmp = us <= uvec
    e_of = jnp.sum(cmp.astype(jnp.float32), axis=1, keepdims=True) - 1.0
    eidw = jax.lax.broadcasted_iota(jnp.int32, (wpad, e), 1).astype(jnp.float32)
    ohw = (eidw == e_of).astype(jnp.float32)
    first_u = jnp.sum(ohw * first, axis=1, keepdims=True)
    us_u = jnp.sum(ohw * us, axis=1, keepdims=True)
    rs_u = jnp.sum(ohw * starts, axis=1, keepdims=True)
    re_u = jnp.sum(ohw * ends, axis=1, keepdims=True)
    ti_u = first_u + (uvec - us_u)
    valid = uvec < total
    lastmask = (uvec == total - 1.0).astype(jnp.float32)
    ti_last = jnp.sum(lastmask * ti_u)
    ei_last = jnp.sum(lastmask * e_of)
    ti_u = jnp.where(valid, ti_u, ti_last)
    e_of = jnp.where(valid, e_of, ei_last)
    rs_u = jnp.where(valid, rs_u, 0.0)
    re_u = jnp.where(valid, re_u, 0.0)
    ti_ref[...] = ti_u.astype(jnp.int32)
    ei_ref[...] = e_of.astype(jnp.int32)
    rs_ref[...] = rs_u.astype(jnp.int32)
    re_ref[...] = re_u.astype(jnp.int32)


def _routing(x, gate_w, wpad):
    t, _ = x.shape
    e = gate_w.shape[0]
    return pl.pallas_call(
        _routing_body,
        out_shape=(
            jax.ShapeDtypeStruct((t, 1), jnp.int32),
            jax.ShapeDtypeStruct((t, 1), jnp.int32),
            jax.ShapeDtypeStruct((t, 1), jnp.float32),
            jax.ShapeDtypeStruct((t, 1), jnp.float32),
            jax.ShapeDtypeStruct((wpad, 1), jnp.int32),
            jax.ShapeDtypeStruct((wpad, 1), jnp.int32),
            jax.ShapeDtypeStruct((wpad, 1), jnp.int32),
            jax.ShapeDtypeStruct((wpad, 1), jnp.int32),
        ),
    )(x, gate_w)


def _gmm_body(ti_ref, ei_ref, rs_ref, re_ref, xs_ref, w1_ref, w2_ref, w3_ref, out_ref):
    u = pl.program_id(0)
    hj = pl.program_id(1)
    rs = rs_ref[u]
    re = re_ref[u]

    @pl.when(re > rs)
    def _():
        row0 = ti_ref[u] * TM
        gid = row0 + jax.lax.broadcasted_iota(jnp.int32, (TM, 1), 0)
        mask = (gid >= rs) & (gid < re)
        x_t = xs_ref[...].astype(jnp.bfloat16)
        w1c = w1_ref[0].astype(jnp.bfloat16)
        w3c = w3_ref[0].astype(jnp.bfloat16)
        w2c = w2_ref[0].astype(jnp.bfloat16)
        xw1 = jax.lax.dot_general(
            x_t, w1c, (((1,), (1,)), ((), ())), preferred_element_type=jnp.float32
        )
        xw3 = jax.lax.dot_general(
            x_t, w3c, (((1,), (1,)), ((), ())), preferred_element_type=jnp.float32
        )
        h = (xw1 * jax.nn.sigmoid(xw1)) * xw3
        o = jax.lax.dot_general(
            h.astype(jnp.bfloat16), w2c,
            (((1,), (0,)), ((), ())), preferred_element_type=jnp.float32
        )
        prev = out_ref[...]

        @pl.when(hj == 0)
        def _():
            out_ref[...] = jnp.where(mask, o, prev)

        @pl.when(hj != 0)
        def _():
            out_ref[...] = jnp.where(mask, prev + o, prev)


def _gmm(xs, w1, w2, w3, ti, ei, rs, re, num_units):
    s, d = xs.shape
    e, h, _ = w1.shape
    nhj = h // BH
    wspec = pl.BlockSpec((1, BH, d), lambda u, hj, ti, ei, rs, re: (ei[u], hj, 0))
    xspec = pl.BlockSpec((TM, d), lambda u, hj, ti, ei, rs, re: (ti[u], 0))
    ospec = pl.BlockSpec((TM, d), lambda u, hj, ti, ei, rs, re: (ti[u], 0))
    return pl.pallas_call(
        _gmm_body,
        grid_spec=pltpu.PrefetchScalarGridSpec(
            num_scalar_prefetch=4,
            grid=(num_units, nhj),
            in_specs=[xspec, wspec, wspec, wspec],
            out_specs=ospec,
        ),
        out_shape=jax.ShapeDtypeStruct((s, d), jnp.float32),
    )(ti, ei, rs, re, xs, w1, w2, w3)


def _combine_body(g0_ref, g1_ref, w0_ref, w1_ref, y_ref):
    y_ref[...] = g0_ref[...] * w0_ref[...] + g1_ref[...] * w1_ref[...]


def _combine(g0, g1, w0, w1):
    t, d = g0.shape
    tmc = 256
    return pl.pallas_call(
        _combine_body,
        grid=(t // tmc,),
        in_specs=[
            pl.BlockSpec((tmc, d), lambda i: (i, 0)),
            pl.BlockSpec((tmc, d), lambda i: (i, 0)),
            pl.BlockSpec((tmc, 1), lambda i: (i, 0)),
            pl.BlockSpec((tmc, 1), lambda i: (i, 0)),
        ],
        out_specs=pl.BlockSpec((tmc, d), lambda i: (i, 0)),
        out_shape=jax.ShapeDtypeStruct((t, d), jnp.float32),
    )(g0, g1, w0, w1)


_NC = 2
_NS = 16
_NW = _NC * _NS


def _sc_dispatch(x, p0, p1, s):
    t, d = x.shape
    bpw = t // _NW
    mesh = plsc.VectorSubcoreMesh(core_axis_name="c", subcore_axis_name="s")

    @functools.partial(
        pl.kernel,
        mesh=mesh,
        out_type=jax.ShapeDtypeStruct((s, d), jnp.float32),
        scratch_types=[
            pltpu.VMEM((bpw,), jnp.int32),
            pltpu.VMEM((bpw,), jnp.int32),
            pltpu.VMEM((bpw, d), jnp.float32),
            pltpu.SemaphoreType.DMA,
        ],
    )
    def k(x_hbm, p0_hbm, p1_hbm, o_hbm, i0_v, i1_v, rows_v, sem):
        wid = jax.lax.axis_index("s") * _NC + jax.lax.axis_index("c")
        base = wid * bpw
        pltpu.sync_copy(p0_hbm.at[pl.ds(base, bpw)], i0_v)
        pltpu.sync_copy(p1_hbm.at[pl.ds(base, bpw)], i1_v)
        pltpu.sync_copy(x_hbm.at[pl.ds(base, bpw)], rows_v)
        pltpu.async_copy(rows_v, o_hbm.at[i0_v], sem).wait()
        pltpu.async_copy(rows_v, o_hbm.at[i1_v], sem).wait()

    return k(x, p0, p1)


def _sc_gather2(os_, p0, p1):
    s, d = os_.shape
    t = p0.shape[0]
    bpw = t // _NW
    mesh = plsc.VectorSubcoreMesh(core_axis_name="c", subcore_axis_name="s")
    ot = jax.ShapeDtypeStruct((t, d), jnp.float32)

    @functools.partial(
        pl.kernel,
        mesh=mesh,
        out_type=(ot, ot),
        scratch_types=[
            pltpu.VMEM((bpw,), jnp.int32),
            pltpu.VMEM((bpw, d), jnp.float32),
            pltpu.SemaphoreType.DMA,
        ],
    )
    def k(os_hbm, p0_hbm, p1_hbm, g0_hbm, g1_hbm, idx_v, rows_v, sem):
        wid = jax.lax.axis_index("s") * _NC + jax.lax.axis_index("c")
        base = wid * bpw
        pltpu.sync_copy(p0_hbm.at[pl.ds(base, bpw)], idx_v)
        pltpu.async_copy(os_hbm.at[idx_v], rows_v, sem).wait()
        pltpu.sync_copy(rows_v, g0_hbm.at[pl.ds(base, bpw)])
        pltpu.sync_copy(p1_hbm.at[pl.ds(base, bpw)], idx_v)
        pltpu.async_copy(os_hbm.at[idx_v], rows_v, sem).wait()
        pltpu.sync_copy(rows_v, g1_hbm.at[pl.ds(base, bpw)])

    return k(os_, p0, p1)


def kernel(x, gate_w, w1, w2, w3):
    t, d = x.shape
    e = gate_w.shape[0]
    k = 2
    s = t * k
    num_tiles = s // TM
    w_static = num_tiles + e - 1

    wpad = -(-w_static // 8) * 8

    pos0, pos1, wt0, wt1, ti2, ei2, rs2, re2 = _routing(x, gate_w, wpad)
    p0r = pos0.reshape(t)
    p1r = pos1.reshape(t)
    ti = ti2.reshape(wpad)[:w_static]
    ei = ei2.reshape(wpad)[:w_static]
    rs = rs2.reshape(wpad)[:w_static]
    re = re2.reshape(wpad)[:w_static]

    xs = _sc_dispatch(x, p0r, p1r, s)
    os = _gmm(xs, w1, w2, w3, ti, ei, rs, re, w_static)
    g0, g1 = _sc_gather2(os, p0r, p1r)
    return _combine(g0, g1, wt0, wt1)
